# Initial kernel scaffold; baseline (speedup 1.0000x reference)
import jax
import jax.numpy as jnp
from jax.experimental import pallas as pl


def kernel(x, conv1_w, conv1_b, bn1_g, bn1_b, bn1_m, bn1_v, conv2_w, conv2_b, bn2_g, bn2_b, bn2_m, bn2_v, sp_w1, sp_b1, sp_w2, sp_b2, tmc_w1, tmc_b1, tmc_w2, tmc_b2, tmm_w1, tmm_b1, tmm_w2, tmm_b2, post_w1, post_b1, post_w2, post_b2):
    raise NotImplementedError("write your pallas kernel here")


# speedup vs baseline: 1.3687x; 1.3687x over previous
import jax
import jax.numpy as jnp
from jax.experimental import pallas as pl
from jax.experimental.pallas import tpu as pltpu

F32 = jnp.float32
B, PRIOR, FRAMES, POSE, PRED, CHUNK = 256, 60, 240, 768, 180, 10
EPS = 1e-5

BB_CONV = 8
BB_POST = 8


def _conv_body(x_ref, w1_ref, w2_ref, cb1_ref, s1_ref, t1_ref,
               cb2_ref, s2_ref, t2_ref, p_ref):
    for b in range(BB_CONV):
        xb = x_ref[b]
        z1 = jnp.zeros((PRIOR, 1), F32)
        xm = jnp.concatenate([z1, xb[:, :-1]], axis=1)
        xp = jnp.concatenate([xb[:, 1:], z1], axis=1)
        x3 = jnp.concatenate([xm, xb, xp], axis=0)
        c1 = jnp.dot(w1_ref[...], x3, preferred_element_type=F32)
        h1 = jnp.maximum(c1 + cb1_ref[...], 0.0) * s1_ref[...] + t1_ref[...]
        z2 = jnp.zeros((PRED, 1), F32)
        hm = jnp.concatenate([z2, h1[:, :-1]], axis=1)
        hp = jnp.concatenate([h1[:, 1:], z2], axis=1)
        h3 = jnp.concatenate([hm, h1, hp], axis=0)
        c2 = jnp.dot(w2_ref[...], h3, preferred_element_type=F32)
        p_ref[b] = jnp.maximum(c2 + cb2_ref[...], 0.0) * s2_ref[...] + t2_ref[...]


def _mem_body(xt_ref, spw1_ref, tmw1_ref, spb1_ref, spw2t_ref, spb2_ref,
              tmb1_ref, tmw2t_ref, tmb2_ref, mem_ref, mem2_ref, acc1, acc2):
    c = pl.program_id(0)

    @pl.when(c == 0)
    def _():
        acc1[...] = jnp.zeros_like(acc1)
        acc2[...] = jnp.zeros_like(acc2)

    xc = xt_ref[0]
    acc1[...] += jnp.dot(xc, spw1_ref[0], preferred_element_type=F32)
    acc2[...] += jnp.dot(xc, tmw1_ref[0], preferred_element_type=F32)

    @pl.when(c == CHUNK - 1)
    def _():
        m1 = acc1[...] + spb1_ref[...]
        mem_ref[...] = (jnp.dot(m1, spw2t_ref[...], preferred_element_type=F32)
                        + spb2_ref[...])
        m2 = acc2[...] + tmb1_ref[...]
        mem2_ref[...] = (jnp.dot(m2, tmw2t_ref[...], preferred_element_type=F32)
                         + tmb2_ref[...])


def _gate_body(pc_ref, mem_ref, mem2_ref, tmmw1_ref, tmmb1_ref,
               tmmw2t_ref, tmmb2_ref, out_ref):
    mem = mem_ref[...]
    mem2 = mem2_ref[...]
    csps = []
    for c in range(CHUNK):
        pcc = pc_ref[:, c, :]
        sc = jnp.sum(mem * pcc, axis=1, keepdims=True)
        sig = jax.nn.sigmoid(sc)
        csps.append(sig * pcc + (1.0 - sig) * mem)
    acc = jnp.zeros((B, CHUNK), F32)
    for c in range(CHUNK):
        acc = acc + jnp.dot(csps[c], tmmw1_ref[c],
                            preferred_element_type=F32)
    penc = (jnp.dot(acc + tmmb1_ref[...], tmmw2t_ref[...],
                    preferred_element_type=F32) + tmmb2_ref[...])
    mmat = jax.lax.dot_general(mem2, penc, (((0,), (0,)), ((), ())),
                               preferred_element_type=F32)
    score2 = jnp.dot(mem2, mmat, preferred_element_type=F32)
    score2 = score2 - jnp.max(score2, axis=1, keepdims=True)
    es = jnp.exp(score2)
    soft = es / jnp.sum(es, axis=1, keepdims=True)
    for c in range(CHUNK):
        out_ref[:, c, :] = csps[c] * (1.0 + soft[:, c:c + 1])


def _post_body(x_ref, p_ref, ch_ref, w1t_ref, b1_ref, w2t_ref, b2_ref, o_ref):
    for b in range(BB_POST):
        rows = jnp.concatenate([x_ref[b], ch_ref[b], p_ref[b, CHUNK:, :]],
                               axis=0)
        h = jnp.dot(rows, w1t_ref[...], preferred_element_type=F32) + b1_ref[...]
        o_ref[b] = jnp.dot(h, w2t_ref[...], preferred_element_type=F32) + b2_ref[...]


def kernel(x, conv1_w, conv1_b, bn1_g, bn1_b, bn1_m, bn1_v,
           conv2_w, conv2_b, bn2_g, bn2_b, bn2_m, bn2_v,
           sp_w1, sp_b1, sp_w2, sp_b2,
           tmc_w1, tmc_b1, tmc_w2, tmc_b2,
           tmm_w1, tmm_b1, tmm_w2, tmm_b2,
           post_w1, post_b1, post_w2, post_b2):
    w1f = conv1_w.transpose(0, 2, 1).reshape(PRED, 3 * PRIOR)
    w2f = conv2_w.transpose(0, 2, 1).reshape(PRED, 3 * PRED)
    s1 = bn1_g * jax.lax.rsqrt(bn1_v + EPS)
    t1 = bn1_b - bn1_m * s1
    s2 = bn2_g * jax.lax.rsqrt(bn2_v + EPS)
    t2 = bn2_b - bn2_m * s2
    bc = lambda v: jnp.broadcast_to(v[:, None], (PRED, POSE))
    cb1, s1b, t1b = bc(conv1_b), bc(s1), bc(t1)
    cb2, s2b, t2b = bc(conv2_b), bc(s2), bc(t2)

    full2 = lambda shape: pl.BlockSpec(shape, lambda i: (0, 0))
    p = pl.pallas_call(
        _conv_body,
        grid=(B // BB_CONV,),
        in_specs=[
            pl.BlockSpec((BB_CONV, PRIOR, POSE), lambda i: (i, 0, 0)),
            full2((PRED, 3 * PRIOR)), full2((PRED, 3 * PRED)),
            full2((PRED, POSE)), full2((PRED, POSE)), full2((PRED, POSE)),
            full2((PRED, POSE)), full2((PRED, POSE)), full2((PRED, POSE)),
        ],
        out_specs=pl.BlockSpec((BB_CONV, PRED, POSE), lambda i: (i, 0, 0)),
        out_shape=jax.ShapeDtypeStruct((B, PRED, POSE), F32),
        compiler_params=pltpu.CompilerParams(
            dimension_semantics=("arbitrary",),
            vmem_limit_bytes=100 * 1024 * 1024,
        ),
        name="conv_encoder",
    )(x, w1f, w2f, cb1, s1b, t1b, cb2, s2b, t2b)

    xt = x[:, PRIOR - CHUNK:, :].transpose(1, 0, 2)
    spw1r = sp_w1.reshape(POSE, CHUNK, POSE).transpose(1, 2, 0)
    tmw1r = tmc_w1.reshape(POSE, CHUNK, POSE).transpose(1, 2, 0)
    mem, mem2 = pl.pallas_call(
        _mem_body,
        grid=(CHUNK,),
        in_specs=[
            pl.BlockSpec((1, B, POSE), lambda c: (c, 0, 0)),
            pl.BlockSpec((1, POSE, POSE), lambda c: (c, 0, 0)),
            pl.BlockSpec((1, POSE, POSE), lambda c: (c, 0, 0)),
            pl.BlockSpec((1, POSE), lambda c: (0, 0)),
            pl.BlockSpec((POSE, POSE), lambda c: (0, 0)),
            pl.BlockSpec((1, POSE), lambda c: (0, 0)),
            pl.BlockSpec((1, POSE), lambda c: (0, 0)),
            pl.BlockSpec((POSE, POSE), lambda c: (0, 0)),
            pl.BlockSpec((1, POSE), lambda c: (0, 0)),
        ],
        out_specs=[
            pl.BlockSpec((B, POSE), lambda c: (0, 0)),
            pl.BlockSpec((B, POSE), lambda c: (0, 0)),
        ],
        out_shape=[
            jax.ShapeDtypeStruct((B, POSE), F32),
            jax.ShapeDtypeStruct((B, POSE), F32),
        ],
        scratch_shapes=[
            pltpu.VMEM((B, POSE), F32),
            pltpu.VMEM((B, POSE), F32),
        ],
        compiler_params=pltpu.CompilerParams(
            dimension_semantics=("arbitrary",),
            vmem_limit_bytes=100 * 1024 * 1024,
        ),
        name="tail_mem",
    )(xt, spw1r, tmw1r, sp_b1.reshape(1, POSE), sp_w2.T,
      sp_b2.reshape(1, POSE), tmc_b1.reshape(1, POSE), tmc_w2.T,
      tmc_b2.reshape(1, POSE))

    tmmw1r = tmm_w1.reshape(CHUNK, CHUNK, POSE).transpose(1, 2, 0)
    chunk_out = pl.pallas_call(
        _gate_body,
        grid=(1,),
        in_specs=[
            pl.BlockSpec((B, 16, POSE), lambda i: (0, 0, 0)),
            pl.BlockSpec((B, POSE), lambda i: (0, 0)),
            pl.BlockSpec((B, POSE), lambda i: (0, 0)),
            pl.BlockSpec((CHUNK, POSE, CHUNK), lambda i: (0, 0, 0)),
            pl.BlockSpec((1, CHUNK), lambda i: (0, 0)),
            pl.BlockSpec((CHUNK, CHUNK), lambda i: (0, 0)),
            pl.BlockSpec((1, CHUNK), lambda i: (0, 0)),
        ],
        out_specs=pl.BlockSpec((B, CHUNK, POSE), lambda i: (0, 0, 0)),
        out_shape=jax.ShapeDtypeStruct((B, CHUNK, POSE), F32),
        compiler_params=pltpu.CompilerParams(
            dimension_semantics=("arbitrary",),
            vmem_limit_bytes=100 * 1024 * 1024,
        ),
        name="gating",
    )(p, mem, mem2, tmmw1r, tmm_b1.reshape(1, CHUNK), tmm_w2.T,
      tmm_b2.reshape(1, CHUNK))

    out = pl.pallas_call(
        _post_body,
        grid=(B // BB_POST,),
        in_specs=[
            pl.BlockSpec((BB_POST, PRIOR, POSE), lambda i: (i, 0, 0)),
            pl.BlockSpec((BB_POST, PRED, POSE), lambda i: (i, 0, 0)),
            pl.BlockSpec((BB_POST, CHUNK, POSE), lambda i: (i, 0, 0)),
            pl.BlockSpec((POSE, POSE), lambda i: (0, 0)),
            pl.BlockSpec((1, POSE), lambda i: (0, 0)),
            pl.BlockSpec((POSE, POSE), lambda i: (0, 0)),
            pl.BlockSpec((1, POSE), lambda i: (0, 0)),
        ],
        out_specs=pl.BlockSpec((BB_POST, FRAMES, POSE), lambda i: (i, 0, 0)),
        out_shape=jax.ShapeDtypeStruct((B, FRAMES, POSE), F32),
        compiler_params=pltpu.CompilerParams(
            dimension_semantics=("arbitrary",),
            vmem_limit_bytes=100 * 1024 * 1024,
        ),
        name="post_header",
    )(x, p, chunk_out, post_w1.T, post_b1.reshape(1, POSE), post_w2.T,
      post_b2.reshape(1, POSE))
    return out
